# Initial kernel scaffold; baseline (speedup 1.0000x reference)
import jax
import jax.numpy as jnp
from jax.experimental import pallas as pl


def kernel(indices, tables):
    raise NotImplementedError("write your pallas kernel here")


# speedup vs baseline: 1.1360x; 1.1360x over previous
import functools

import jax
import jax.numpy as jnp
from jax import lax
from jax.experimental import pallas as pl
from jax.experimental.pallas import tpu as pltpu
from jax.experimental.pallas import tpu_sc as plsc

_NC = 2
_NS = 16
_NW = _NC * _NS
_LANES = 16


def _build(B, F, V, D):
    N = B * F
    CHUNK = 128
    rows_w = N // _NW
    ch_w = rows_w // CHUNK
    MEGA = 8
    n_mega = ch_w // MEGA
    PERIOD = 13
    assert rows_w % CHUNK == 0 and ch_w % MEGA == 0
    assert (PERIOD * CHUNK) % F == 0 and ch_w % PERIOD == 0
    VECS = CHUNK // _LANES

    mesh = plsc.VectorSubcoreMesh(core_axis_name="c", subcore_axis_name="s")

    @functools.partial(
        pl.kernel,
        out_type=jax.ShapeDtypeStruct((N, D), jnp.float32),
        mesh=mesh,
        compiler_params=pltpu.CompilerParams(use_tc_tiling_on_sc=False),
        scratch_types=[
            pltpu.VMEM((MEGA, CHUNK), jnp.int32),
            pltpu.VMEM((PERIOD, CHUNK), jnp.int32),
            pltpu.VMEM((MEGA * CHUNK, D), jnp.float32),
            pltpu.SemaphoreType.DMA,
        ],
    )
    def k(idx_hbm, tab_hbm, out_hbm, idx_v, offs_v, rows_v, sem):
        wid = lax.axis_index("s") * _NC + lax.axis_index("c")

        f = lax.iota(jnp.int32, _LANES)
        for t in range(PERIOD):
            for v in range(VECS):
                offs_v[t, pl.ds(v * _LANES, _LANES)] = f * V
                f = f + _LANES
                f = f - jnp.where(f >= F, F, 0)

        for m in range(n_mega):
            base_chunk = pl.multiple_of(wid * ch_w + m * MEGA, MEGA)
            pltpu.sync_copy(idx_hbm.at[pl.ds(base_chunk, MEGA)], idx_v)
            for j in range(MEGA):
                t = (m * MEGA + j) % PERIOD
                for v in range(VECS):
                    sl = pl.ds(v * _LANES, _LANES)
                    idx_v[j, sl] = idx_v[j, sl] + offs_v[t, sl]
            copies = [
                pltpu.async_copy(
                    tab_hbm.at[idx_v.at[j]],
                    rows_v.at[pl.ds(j * CHUNK, CHUNK)],
                    sem,
                )
                for j in range(MEGA)
            ]
            for c in copies:
                c.wait()
            pltpu.sync_copy(
                rows_v, out_hbm.at[pl.ds(base_chunk * CHUNK, MEGA * CHUNK)]
            )

    return k


def kernel(indices, tables):
    B, F = indices.shape
    _, V, D = tables.shape
    idx_flat = indices.reshape(B * F // 128, 128)
    tab_flat = tables.reshape(F * V, D)
    out = _build(B, F, V, D)(idx_flat, tab_flat)
    return out.reshape(B, F * D)
